# baseline (speedup 1.0000x reference)
import jax
import jax.numpy as jnp
from jax import lax
from jax.experimental import pallas as pl
from jax.experimental.pallas import tpu as pltpu
from jax.experimental.pallas import tpu_sc as plsc

B, S, H = 1, 2048, 1024
E, K, INTER = 8, 2, 1024
ALPHA = 1.702
LIMIT = 7.0
T = B * S
CH = 256
NW = 32
TPW = T // NW
IPW = TPW * K
TM = 256
NT = 24
ND = NT * TM
NMETA = 48


def _router_body(x_ref, wr_ref, rb_ref, w1_ref, w2_ref, p1_ref, p2_ref,
                 meta_ref, val_ref):
    logits = lax.dot_general(
        wr_ref[...], x_ref[...], (((1,), (1,)), ((), ())),
        preferred_element_type=jnp.float32) + rb_ref[...]
    m = jnp.max(logits, axis=0, keepdims=True)
    p = jnp.exp(logits - m)
    s = p / jnp.sum(p, axis=0, keepdims=True)
    iota_e = lax.broadcasted_iota(jnp.int32, (E, T), 0)
    m1 = jnp.max(s, axis=0, keepdims=True)
    idx1 = jnp.min(jnp.where(s == m1, iota_e, E), axis=0, keepdims=True)
    not1 = iota_e != idx1
    m2 = jnp.max(jnp.where(not1, s, -jnp.inf), axis=0, keepdims=True)
    idx2 = jnp.min(jnp.where(not1 & (s == m2), iota_e, E), axis=0,
                   keepdims=True)
    w1_ref[...] = m1
    w2_ref[...] = m2

    oh1 = (iota_e == idx1).astype(jnp.float32)
    oh2 = (iota_e == idx2).astype(jnp.float32)
    oh12 = oh1 + oh2
    tot_c = jnp.sum(oh12, axis=1, keepdims=True)
    eye = (lax.broadcasted_iota(jnp.int32, (E, E), 0) ==
           lax.broadcasted_iota(jnp.int32, (E, E), 1)).astype(jnp.float32)
    tot = jnp.sum(tot_c * eye, axis=0, keepdims=True)

    ntiles = jnp.floor((tot + (TM - 1)) * (1.0 / TM))
    shift = (lax.broadcasted_iota(jnp.int32, (E, E), 0) <
             lax.broadcasted_iota(jnp.int32, (E, E), 1)).astype(jnp.float32)
    ts = jnp.dot(ntiles, shift,
                 preferred_element_type=jnp.float32)
    base = ts * TM

    tt = lax.broadcasted_iota(jnp.int32, (NMETA, E), 0).astype(jnp.float32)
    cmp = (tt >= ts).astype(jnp.float32)
    me = jnp.sum(cmp, axis=1, keepdims=True) - 1.0
    oh_m = (lax.broadcasted_iota(jnp.int32, (NMETA, E), 1).astype(jnp.float32)
            == me)
    tot_sel = jnp.sum(jnp.where(oh_m, tot, 0.0), axis=1, keepdims=True)
    ts_sel = jnp.sum(jnp.where(oh_m, ts, 0.0), axis=1, keepdims=True)
    tloc = lax.broadcasted_iota(jnp.int32, (NMETA, 1), 0).astype(jnp.float32)
    vcnt = jnp.clip(tot_sel - (tloc - ts_sel) * TM, 0.0, TM)
    meta_ref[...] = me.astype(jnp.int32)
    val_ref[...] = vcnt.astype(jnp.int32)

    ltri = (lax.broadcasted_iota(jnp.int32, (T, T), 0) <
            lax.broadcasted_iota(jnp.int32, (T, T), 1)).astype(jnp.float32)
    pexc = jnp.dot(oh12, ltri, preferred_element_type=jnp.float32)
    base_c = jnp.sum(base * eye, axis=1, keepdims=True)
    off = base_c + pexc
    pos1 = jnp.sum(oh1 * off, axis=0, keepdims=True)
    pos2 = jnp.sum(oh2 * (off + oh1), axis=0, keepdims=True)
    p1_ref[...] = pos1.astype(jnp.int32)
    p2_ref[...] = pos2.astype(jnp.int32)


def _router(flat, router_weight, rb2):
    return pl.pallas_call(
        _router_body,
        in_specs=[
            pl.BlockSpec((T, H), lambda: (0, 0)),
            pl.BlockSpec((E, H), lambda: (0, 0)),
            pl.BlockSpec((E, 1), lambda: (0, 0)),
        ],
        out_specs=[
            pl.BlockSpec((1, T), lambda: (0, 0)),
            pl.BlockSpec((1, T), lambda: (0, 0)),
            pl.BlockSpec((1, T), lambda: (0, 0)),
            pl.BlockSpec((1, T), lambda: (0, 0)),
            pl.BlockSpec((NMETA, 1), lambda: (0, 0)),
            pl.BlockSpec((NMETA, 1), lambda: (0, 0)),
        ],
        out_shape=[
            jax.ShapeDtypeStruct((1, T), jnp.float32),
            jax.ShapeDtypeStruct((1, T), jnp.float32),
            jax.ShapeDtypeStruct((1, T), jnp.int32),
            jax.ShapeDtypeStruct((1, T), jnp.int32),
            jax.ShapeDtypeStruct((NMETA, 1), jnp.int32),
            jax.ShapeDtypeStruct((NMETA, 1), jnp.int32),
        ],
    )(flat, router_weight, rb2)


def _sc_dispatch_body(p1_hbm, p2_hbm, x_hbm, xd_hbm, pe_v, po_v, rows_v,
                      sem):
    wid = lax.axis_index("s") * 2 + lax.axis_index("c")
    pltpu.sync_copy(p1_hbm.at[wid], pe_v.at[0])
    pltpu.sync_copy(p2_hbm.at[wid], po_v.at[0])
    pltpu.sync_copy(x_hbm.at[pl.ds(wid * TPW, TPW)], rows_v)
    cp1 = pltpu.async_copy(rows_v, xd_hbm.at[pe_v.at[0]], sem)
    cp2 = pltpu.async_copy(rows_v, xd_hbm.at[po_v.at[0]], sem)
    cp1.wait()
    cp2.wait()


def _sc_dispatch(p1, p2, flat):
    return pl.kernel(
        _sc_dispatch_body,
        out_type=jax.ShapeDtypeStruct((ND, H), jnp.float32),
        mesh=plsc.VectorSubcoreMesh(core_axis_name="c", subcore_axis_name="s",
                                    num_cores=2, num_subcores=16),
        scratch_types=[
            pltpu.VMEM((1, TPW), jnp.int32),
            pltpu.VMEM((1, TPW), jnp.int32),
            pltpu.VMEM((TPW, H), jnp.float32),
            pltpu.SemaphoreType.DMA,
        ],
    )(p1, p2, flat)


def _grouped_body(meta_ref, val_ref, xd_ref, wg_ref, wu_ref, gub_ref,
                  wd1_ref, wd2_ref, db_ref, yd_ref):
    t = pl.program_id(0)

    @pl.when(val_ref[t] > 0)
    def _():
        xt = xd_ref[...]
        gate = jnp.dot(xt, wg_ref[0],
                       preferred_element_type=jnp.float32) + gub_ref[0, :, :INTER]
        up = jnp.dot(xt, wu_ref[0],
                     preferred_element_type=jnp.float32) + gub_ref[0, :, INTER:]
        gate = jnp.minimum(gate, LIMIT)
        up = jnp.clip(up, -LIMIT, LIMIT)
        act = (up + 1.0) * (gate * jax.nn.sigmoid(gate * ALPHA))
        hi = INTER // 2
        yd_ref[...] = (jnp.dot(act[:, :hi], wd1_ref[0],
                               preferred_element_type=jnp.float32) +
                       jnp.dot(act[:, hi:], wd2_ref[0],
                               preferred_element_type=jnp.float32) +
                       db_ref[0])


def _grouped(meta, valid, xd, wgu, gub3, wd, db3):
    grid_spec = pltpu.PrefetchScalarGridSpec(
        num_scalar_prefetch=2,
        grid=(NT,),
        in_specs=[
            pl.BlockSpec((TM, H), lambda t, m, v: (t, 0)),
            pl.BlockSpec((1, H, INTER), lambda t, m, v: (m[t], 0, 0)),
            pl.BlockSpec((1, H, INTER), lambda t, m, v: (m[t], 0, 1)),
            pl.BlockSpec((1, 1, 2 * INTER), lambda t, m, v: (m[t], 0, 0)),
            pl.BlockSpec((1, INTER // 2, H), lambda t, m, v: (m[t], 0, 0)),
            pl.BlockSpec((1, INTER // 2, H), lambda t, m, v: (m[t], 1, 0)),
            pl.BlockSpec((1, 1, H), lambda t, m, v: (m[t], 0, 0)),
        ],
        out_specs=pl.BlockSpec((TM, H), lambda t, m, v: (t, 0)),
    )
    return pl.pallas_call(
        _grouped_body,
        grid_spec=grid_spec,
        out_shape=jax.ShapeDtypeStruct((ND, H), jnp.float32),
        compiler_params=pltpu.CompilerParams(
            dimension_semantics=("arbitrary",)),
    )(meta, valid, xd, wgu, wgu, gub3, wd, wd, db3)


_CC = 16
_NC = TPW // _CC


def _sc_combine_body(yd_hbm, p1_hbm, p2_hbm, w1_hbm, w2_hbm, out_hbm,
                     pc1_v, pc2_v, wc1_v, wc2_v, rows1_v, rows2_v, obuf_v,
                     sem1, sem2):
    wid = lax.axis_index("s") * 2 + lax.axis_index("c")
    pltpu.sync_copy(p1_hbm.at[pl.ds(wid * _NC, _NC)], pc1_v)
    pltpu.sync_copy(p2_hbm.at[pl.ds(wid * _NC, _NC)], pc2_v)
    pltpu.sync_copy(w1_hbm.at[pl.ds(wid * _NC, _NC)], wc1_v)
    pltpu.sync_copy(w2_hbm.at[pl.ds(wid * _NC, _NC)], wc2_v)
    cps = {}
    for c in range(2):
        cps[c] = (pltpu.async_copy(yd_hbm.at[pc1_v.at[c]],
                                   rows1_v.at[c % 2], sem1),
                  pltpu.async_copy(yd_hbm.at[pc2_v.at[c]],
                                   rows2_v.at[c % 2], sem2))
    for c in range(_NC):
        b = c % 2
        cps[c][0].wait()
        cps[c][1].wait()
        wr1 = wc1_v[c, :]
        wr2 = wc2_v[c, :]
        for j in range(_CC):
            w1v = jnp.full((16,), wr1[j], jnp.float32)
            w2v = jnp.full((16,), wr2[j], jnp.float32)

            def body(i, _, b=b, j=j, w1v=w1v, w2v=w2v):
                for u in range(8):
                    sl = pl.ds(i * 128 + u * 16, 16)
                    obuf_v[j, sl] = (w1v * rows1_v[b, j, sl] +
                                     w2v * rows2_v[b, j, sl])
                return 0

            lax.fori_loop(0, H // 128, body, 0)
        pltpu.sync_copy(obuf_v,
                        out_hbm.at[pl.ds(wid * TPW + c * _CC, _CC)])
        if c + 2 < _NC:
            cps[c + 2] = (pltpu.async_copy(yd_hbm.at[pc1_v.at[c + 2]],
                                           rows1_v.at[b], sem1),
                          pltpu.async_copy(yd_hbm.at[pc2_v.at[c + 2]],
                                           rows2_v.at[b], sem2))


def _sc_combine(yd, p1, p2, w1, w2):
    return pl.kernel(
        _sc_combine_body,
        out_type=jax.ShapeDtypeStruct((T, H), jnp.float32),
        mesh=plsc.VectorSubcoreMesh(core_axis_name="c", subcore_axis_name="s",
                                    num_cores=2, num_subcores=16),
        scratch_types=[
            pltpu.VMEM((_NC, _CC), jnp.int32),
            pltpu.VMEM((_NC, _CC), jnp.int32),
            pltpu.VMEM((_NC, _CC), jnp.float32),
            pltpu.VMEM((_NC, _CC), jnp.float32),
            pltpu.VMEM((2, _CC, H), jnp.float32),
            pltpu.VMEM((2, _CC, H), jnp.float32),
            pltpu.VMEM((_CC, H), jnp.float32),
            pltpu.SemaphoreType.DMA,
            pltpu.SemaphoreType.DMA,
        ],
    )(yd, p1, p2, w1, w2)


def kernel(hidden_states, router_weight, router_bias, gate_up_proj,
           gate_up_bias, down_proj, down_bias):
    flat = hidden_states.reshape(T, H)
    rb2 = router_bias.reshape(E, 1)
    w1, w2, p1, p2, meta, valid = _router(flat, router_weight, rb2)
    xd = _sc_dispatch(p1.reshape(NW, TPW), p2.reshape(NW, TPW), flat)
    yd = _grouped(meta.reshape(NMETA), valid.reshape(NMETA), xd,
                  gate_up_proj, gate_up_bias.reshape(E, 1, 2 * INTER),
                  down_proj, down_bias.reshape(E, 1, H))
    out = _sc_combine(yd,
                      p1.reshape(NW * _NC, _CC), p2.reshape(NW * _NC, _CC),
                      w1.reshape(NW * _NC, _CC), w2.reshape(NW * _NC, _CC))
    return out.reshape(B, S, H)


# speedup vs baseline: 1.0079x; 1.0079x over previous
import jax
import jax.numpy as jnp
from jax import lax
from jax.experimental import pallas as pl
from jax.experimental.pallas import tpu as pltpu
from jax.experimental.pallas import tpu_sc as plsc

B, S, H = 1, 2048, 1024
E, K, INTER = 8, 2, 1024
ALPHA = 1.702
LIMIT = 7.0
T = B * S
CH = 256
NW = 32
TPW = T // NW
IPW = TPW * K
TM = 256
NT = 24
ND = NT * TM
NMETA = 48


def _router_body(x_ref, wr_ref, rb_ref, w1_ref, w2_ref, p1_ref, p2_ref,
                 meta_ref, val_ref):
    logits = lax.dot_general(
        wr_ref[...], x_ref[...], (((1,), (1,)), ((), ())),
        preferred_element_type=jnp.float32) + rb_ref[...]
    m = jnp.max(logits, axis=0, keepdims=True)
    p = jnp.exp(logits - m)
    s = p / jnp.sum(p, axis=0, keepdims=True)
    iota_e = lax.broadcasted_iota(jnp.int32, (E, T), 0)
    m1 = jnp.max(s, axis=0, keepdims=True)
    idx1 = jnp.min(jnp.where(s == m1, iota_e, E), axis=0, keepdims=True)
    not1 = iota_e != idx1
    m2 = jnp.max(jnp.where(not1, s, -jnp.inf), axis=0, keepdims=True)
    idx2 = jnp.min(jnp.where(not1 & (s == m2), iota_e, E), axis=0,
                   keepdims=True)
    w1_ref[...] = m1
    w2_ref[...] = m2

    oh1 = (iota_e == idx1).astype(jnp.float32)
    oh2 = (iota_e == idx2).astype(jnp.float32)
    oh12 = oh1 + oh2
    tot_c = jnp.sum(oh12, axis=1, keepdims=True)
    eye = (lax.broadcasted_iota(jnp.int32, (E, E), 0) ==
           lax.broadcasted_iota(jnp.int32, (E, E), 1)).astype(jnp.float32)
    tot = jnp.sum(tot_c * eye, axis=0, keepdims=True)

    ntiles = jnp.floor((tot + (TM - 1)) * (1.0 / TM))
    shift = (lax.broadcasted_iota(jnp.int32, (E, E), 0) <
             lax.broadcasted_iota(jnp.int32, (E, E), 1)).astype(jnp.float32)
    ts = jnp.dot(ntiles, shift,
                 preferred_element_type=jnp.float32)
    base = ts * TM

    tt = lax.broadcasted_iota(jnp.int32, (NMETA, E), 0).astype(jnp.float32)
    cmp = (tt >= ts).astype(jnp.float32)
    me = jnp.sum(cmp, axis=1, keepdims=True) - 1.0
    oh_m = (lax.broadcasted_iota(jnp.int32, (NMETA, E), 1).astype(jnp.float32)
            == me)
    tot_sel = jnp.sum(jnp.where(oh_m, tot, 0.0), axis=1, keepdims=True)
    ts_sel = jnp.sum(jnp.where(oh_m, ts, 0.0), axis=1, keepdims=True)
    tloc = lax.broadcasted_iota(jnp.int32, (NMETA, 1), 0).astype(jnp.float32)
    vcnt = jnp.clip(tot_sel - (tloc - ts_sel) * TM, 0.0, TM)
    meta_ref[...] = me.astype(jnp.int32)
    val_ref[...] = vcnt.astype(jnp.int32)

    ltri = (lax.broadcasted_iota(jnp.int32, (T, T), 0) <
            lax.broadcasted_iota(jnp.int32, (T, T), 1)).astype(jnp.float32)
    pexc = jnp.dot(oh12, ltri, preferred_element_type=jnp.float32)
    base_c = jnp.sum(base * eye, axis=1, keepdims=True)
    off = base_c + pexc
    pos1 = jnp.sum(oh1 * off, axis=0, keepdims=True)
    pos2 = jnp.sum(oh2 * (off + oh1), axis=0, keepdims=True)
    p1_ref[...] = pos1.astype(jnp.int32)
    p2_ref[...] = pos2.astype(jnp.int32)


def _router(flat, router_weight, rb2):
    return pl.pallas_call(
        _router_body,
        in_specs=[
            pl.BlockSpec((T, H), lambda: (0, 0)),
            pl.BlockSpec((E, H), lambda: (0, 0)),
            pl.BlockSpec((E, 1), lambda: (0, 0)),
        ],
        out_specs=[
            pl.BlockSpec((1, T), lambda: (0, 0)),
            pl.BlockSpec((1, T), lambda: (0, 0)),
            pl.BlockSpec((1, T), lambda: (0, 0)),
            pl.BlockSpec((1, T), lambda: (0, 0)),
            pl.BlockSpec((NMETA, 1), lambda: (0, 0)),
            pl.BlockSpec((NMETA, 1), lambda: (0, 0)),
        ],
        out_shape=[
            jax.ShapeDtypeStruct((1, T), jnp.float32),
            jax.ShapeDtypeStruct((1, T), jnp.float32),
            jax.ShapeDtypeStruct((1, T), jnp.int32),
            jax.ShapeDtypeStruct((1, T), jnp.int32),
            jax.ShapeDtypeStruct((NMETA, 1), jnp.int32),
            jax.ShapeDtypeStruct((NMETA, 1), jnp.int32),
        ],
    )(flat, router_weight, rb2)


def _sc_dispatch_body(p1_hbm, p2_hbm, x_hbm, xd_hbm, pe_v, po_v, rows_v,
                      sem):
    wid = lax.axis_index("s") * 2 + lax.axis_index("c")
    pltpu.sync_copy(p1_hbm.at[wid], pe_v.at[0])
    pltpu.sync_copy(p2_hbm.at[wid], po_v.at[0])
    pltpu.sync_copy(x_hbm.at[pl.ds(wid * TPW, TPW)], rows_v)
    cp1 = pltpu.async_copy(rows_v, xd_hbm.at[pe_v.at[0]], sem)
    cp2 = pltpu.async_copy(rows_v, xd_hbm.at[po_v.at[0]], sem)
    cp1.wait()
    cp2.wait()


def _sc_dispatch(p1, p2, flat):
    return pl.kernel(
        _sc_dispatch_body,
        out_type=jax.ShapeDtypeStruct((ND, H), jnp.float32),
        mesh=plsc.VectorSubcoreMesh(core_axis_name="c", subcore_axis_name="s",
                                    num_cores=2, num_subcores=16),
        scratch_types=[
            pltpu.VMEM((1, TPW), jnp.int32),
            pltpu.VMEM((1, TPW), jnp.int32),
            pltpu.VMEM((TPW, H), jnp.float32),
            pltpu.SemaphoreType.DMA,
        ],
    )(p1, p2, flat)


def _grouped_body(meta_ref, val_ref, xd_ref, wgu_ref, gub_ref, wd_ref, db_ref,
                  yd_ref):
    t = pl.program_id(0)

    @pl.when(val_ref[t] > 0)
    def _():
        xt = xd_ref[...]
        gu = jnp.dot(xt, wgu_ref[0],
                     preferred_element_type=jnp.float32) + gub_ref[0]
        gate = jnp.minimum(gu[:, :INTER], LIMIT)
        up = jnp.clip(gu[:, INTER:], -LIMIT, LIMIT)
        act = (up + 1.0) * (gate * jax.nn.sigmoid(gate * ALPHA))
        yd_ref[...] = jnp.dot(act, wd_ref[0],
                              preferred_element_type=jnp.float32) + db_ref[0]


def _grouped(meta, valid, xd, wgu, gub3, wd, db3):
    grid_spec = pltpu.PrefetchScalarGridSpec(
        num_scalar_prefetch=2,
        grid=(NT,),
        in_specs=[
            pl.BlockSpec((TM, H), lambda t, m, v: (t, 0)),
            pl.BlockSpec((1, H, 2 * INTER), lambda t, m, v: (m[t], 0, 0)),
            pl.BlockSpec((1, 1, 2 * INTER), lambda t, m, v: (m[t], 0, 0)),
            pl.BlockSpec((1, INTER, H), lambda t, m, v: (m[t], 0, 0)),
            pl.BlockSpec((1, 1, H), lambda t, m, v: (m[t], 0, 0)),
        ],
        out_specs=pl.BlockSpec((TM, H), lambda t, m, v: (t, 0)),
    )
    return pl.pallas_call(
        _grouped_body,
        grid_spec=grid_spec,
        out_shape=jax.ShapeDtypeStruct((ND, H), jnp.float32),
        compiler_params=pltpu.CompilerParams(
            dimension_semantics=("arbitrary",)),
    )(meta, valid, xd, wgu, gub3, wd, db3)


_CC = 16
_NC = TPW // _CC


def _sc_combine_body(yd_hbm, p1_hbm, p2_hbm, w1_hbm, w2_hbm, out_hbm,
                     pc1_v, pc2_v, wc1_v, wc2_v, rows1_v, rows2_v, obuf_v,
                     sem1, sem2):
    wid = lax.axis_index("s") * 2 + lax.axis_index("c")
    pltpu.sync_copy(p1_hbm.at[pl.ds(wid * _NC, _NC)], pc1_v)
    pltpu.sync_copy(p2_hbm.at[pl.ds(wid * _NC, _NC)], pc2_v)
    pltpu.sync_copy(w1_hbm.at[pl.ds(wid * _NC, _NC)], wc1_v)
    pltpu.sync_copy(w2_hbm.at[pl.ds(wid * _NC, _NC)], wc2_v)
    cps = {}
    for c in range(2):
        cps[c] = (pltpu.async_copy(yd_hbm.at[pc1_v.at[c]],
                                   rows1_v.at[c % 2], sem1),
                  pltpu.async_copy(yd_hbm.at[pc2_v.at[c]],
                                   rows2_v.at[c % 2], sem2))
    for c in range(_NC):
        b = c % 2
        cps[c][0].wait()
        cps[c][1].wait()
        wr1 = wc1_v[c, :]
        wr2 = wc2_v[c, :]
        for j in range(_CC):
            w1v = jnp.full((16,), wr1[j], jnp.float32)
            w2v = jnp.full((16,), wr2[j], jnp.float32)

            def body(i, _, b=b, j=j, w1v=w1v, w2v=w2v):
                for u in range(8):
                    sl = pl.ds(i * 128 + u * 16, 16)
                    obuf_v[j, sl] = (w1v * rows1_v[b, j, sl] +
                                     w2v * rows2_v[b, j, sl])
                return 0

            lax.fori_loop(0, H // 128, body, 0)
        pltpu.sync_copy(obuf_v,
                        out_hbm.at[pl.ds(wid * TPW + c * _CC, _CC)])
        if c + 2 < _NC:
            cps[c + 2] = (pltpu.async_copy(yd_hbm.at[pc1_v.at[c + 2]],
                                           rows1_v.at[b], sem1),
                          pltpu.async_copy(yd_hbm.at[pc2_v.at[c + 2]],
                                           rows2_v.at[b], sem2))


def _sc_combine(yd, p1, p2, w1, w2):
    return pl.kernel(
        _sc_combine_body,
        out_type=jax.ShapeDtypeStruct((T, H), jnp.float32),
        mesh=plsc.VectorSubcoreMesh(core_axis_name="c", subcore_axis_name="s",
                                    num_cores=2, num_subcores=16),
        scratch_types=[
            pltpu.VMEM((_NC, _CC), jnp.int32),
            pltpu.VMEM((_NC, _CC), jnp.int32),
            pltpu.VMEM((_NC, _CC), jnp.float32),
            pltpu.VMEM((_NC, _CC), jnp.float32),
            pltpu.VMEM((2, _CC, H), jnp.float32),
            pltpu.VMEM((2, _CC, H), jnp.float32),
            pltpu.VMEM((_CC, H), jnp.float32),
            pltpu.SemaphoreType.DMA,
            pltpu.SemaphoreType.DMA,
        ],
    )(yd, p1, p2, w1, w2)


def kernel(hidden_states, router_weight, router_bias, gate_up_proj,
           gate_up_bias, down_proj, down_bias):
    flat = hidden_states.reshape(T, H)
    rb2 = router_bias.reshape(E, 1)
    w1, w2, p1, p2, meta, valid = _router(flat, router_weight, rb2)
    xd = _sc_dispatch(p1.reshape(NW, TPW), p2.reshape(NW, TPW), flat)
    yd = _grouped(meta.reshape(NMETA), valid.reshape(NMETA), xd,
                  gate_up_proj, gate_up_bias.reshape(E, 1, 2 * INTER),
                  down_proj, down_bias.reshape(E, 1, H))
    out = _sc_combine(yd,
                      p1.reshape(NW * _NC, _CC), p2.reshape(NW * _NC, _CC),
                      w1.reshape(NW * _NC, _CC), w2.reshape(NW * _NC, _CC))
    return out.reshape(B, S, H)


# speedup vs baseline: 1.0080x; 1.0001x over previous
import jax
import jax.numpy as jnp
from jax import lax
from jax.experimental import pallas as pl
from jax.experimental.pallas import tpu as pltpu
from jax.experimental.pallas import tpu_sc as plsc

B, S, H = 1, 2048, 1024
E, K, INTER = 8, 2, 1024
ALPHA = 1.702
LIMIT = 7.0
T = B * S
NW = 32
TPW = T // NW
TM = 256
NT = 24
ND = NT * TM
NMETA = 48


def _router_body(x_ref, wr_ref, rb_ref, w1_ref, w2_ref, p1_ref, p2_ref,
                 meta_ref, val_ref):
    logits = lax.dot_general(
        wr_ref[...], x_ref[...], (((1,), (1,)), ((), ())),
        preferred_element_type=jnp.float32) + rb_ref[...]
    m = jnp.max(logits, axis=0, keepdims=True)
    p = jnp.exp(logits - m)
    s = p / jnp.sum(p, axis=0, keepdims=True)
    iota_e = lax.broadcasted_iota(jnp.int32, (E, T), 0)
    m1 = jnp.max(s, axis=0, keepdims=True)
    idx1 = jnp.min(jnp.where(s == m1, iota_e, E), axis=0, keepdims=True)
    not1 = iota_e != idx1
    m2 = jnp.max(jnp.where(not1, s, -jnp.inf), axis=0, keepdims=True)
    idx2 = jnp.min(jnp.where(not1 & (s == m2), iota_e, E), axis=0,
                   keepdims=True)
    w1_ref[...] = m1
    w2_ref[...] = m2

    oh1 = (iota_e == idx1).astype(jnp.float32)
    oh2 = (iota_e == idx2).astype(jnp.float32)
    oh12 = oh1 + oh2
    tot_c = jnp.sum(oh12, axis=1, keepdims=True)
    eye = (lax.broadcasted_iota(jnp.int32, (E, E), 0) ==
           lax.broadcasted_iota(jnp.int32, (E, E), 1)).astype(jnp.float32)
    tot = jnp.sum(tot_c * eye, axis=0, keepdims=True)

    ntiles = jnp.floor((tot + (TM - 1)) * (1.0 / TM))
    shift = (lax.broadcasted_iota(jnp.int32, (E, E), 0) <
             lax.broadcasted_iota(jnp.int32, (E, E), 1)).astype(jnp.float32)
    ts = jnp.dot(ntiles, shift,
                 preferred_element_type=jnp.float32)
    base = ts * TM

    tt = lax.broadcasted_iota(jnp.int32, (NMETA, E), 0).astype(jnp.float32)
    cmp = (tt >= ts).astype(jnp.float32)
    me = jnp.sum(cmp, axis=1, keepdims=True) - 1.0
    oh_m = (lax.broadcasted_iota(jnp.int32, (NMETA, E), 1).astype(jnp.float32)
            == me)
    tot_sel = jnp.sum(jnp.where(oh_m, tot, 0.0), axis=1, keepdims=True)
    ts_sel = jnp.sum(jnp.where(oh_m, ts, 0.0), axis=1, keepdims=True)
    tloc = lax.broadcasted_iota(jnp.int32, (NMETA, 1), 0).astype(jnp.float32)
    vcnt = jnp.clip(tot_sel - (tloc - ts_sel) * TM, 0.0, TM)
    meta_ref[...] = me.astype(jnp.int32)
    val_ref[...] = vcnt.astype(jnp.int32)

    ltri = (lax.broadcasted_iota(jnp.int32, (T, T), 0) <
            lax.broadcasted_iota(jnp.int32, (T, T), 1)).astype(jnp.float32)
    pexc = jnp.dot(oh12, ltri, preferred_element_type=jnp.float32)
    base_c = jnp.sum(base * eye, axis=1, keepdims=True)
    off = base_c + pexc
    pos1 = jnp.sum(oh1 * off, axis=0, keepdims=True)
    pos2 = jnp.sum(oh2 * (off + oh1), axis=0, keepdims=True)
    p1_ref[...] = pos1.astype(jnp.int32)
    p2_ref[...] = pos2.astype(jnp.int32)


def _router(flat, router_weight, rb2):
    return pl.pallas_call(
        _router_body,
        in_specs=[
            pl.BlockSpec((T, H), lambda: (0, 0)),
            pl.BlockSpec((E, H), lambda: (0, 0)),
            pl.BlockSpec((E, 1), lambda: (0, 0)),
        ],
        out_specs=[
            pl.BlockSpec((1, T), lambda: (0, 0)),
            pl.BlockSpec((1, T), lambda: (0, 0)),
            pl.BlockSpec((1, T), lambda: (0, 0)),
            pl.BlockSpec((1, T), lambda: (0, 0)),
            pl.BlockSpec((NMETA, 1), lambda: (0, 0)),
            pl.BlockSpec((NMETA, 1), lambda: (0, 0)),
        ],
        out_shape=[
            jax.ShapeDtypeStruct((1, T), jnp.float32),
            jax.ShapeDtypeStruct((1, T), jnp.float32),
            jax.ShapeDtypeStruct((1, T), jnp.int32),
            jax.ShapeDtypeStruct((1, T), jnp.int32),
            jax.ShapeDtypeStruct((NMETA, 1), jnp.int32),
            jax.ShapeDtypeStruct((NMETA, 1), jnp.int32),
        ],
    )(flat, router_weight, rb2)


def _sc_dispatch_body(p1_hbm, p2_hbm, x_hbm, xd_hbm, pe_v, po_v, rows_v,
                      sem):
    wid = lax.axis_index("s") * 2 + lax.axis_index("c")
    pltpu.sync_copy(p1_hbm.at[wid], pe_v.at[0])
    pltpu.sync_copy(p2_hbm.at[wid], po_v.at[0])
    pltpu.sync_copy(x_hbm.at[pl.ds(wid * TPW, TPW)], rows_v)
    cp1 = pltpu.async_copy(rows_v, xd_hbm.at[pe_v.at[0]], sem)
    cp2 = pltpu.async_copy(rows_v, xd_hbm.at[po_v.at[0]], sem)
    cp1.wait()
    cp2.wait()


def _sc_dispatch(p1, p2, flat):
    return pl.kernel(
        _sc_dispatch_body,
        out_type=jax.ShapeDtypeStruct((ND, H), jnp.float32),
        mesh=plsc.VectorSubcoreMesh(core_axis_name="c", subcore_axis_name="s",
                                    num_cores=2, num_subcores=16),
        scratch_types=[
            pltpu.VMEM((1, TPW), jnp.int32),
            pltpu.VMEM((1, TPW), jnp.int32),
            pltpu.VMEM((TPW, H), jnp.float32),
            pltpu.SemaphoreType.DMA,
        ],
    )(p1, p2, flat)


def _grouped_body(meta_ref, val_ref, xd_ref, wgu_ref, gub_ref, wd_ref, db_ref,
                  yd_ref):
    t = pl.program_id(0)

    @pl.when(val_ref[t] > 0)
    def _():
        xt = xd_ref[...]
        gu = jnp.dot(xt, wgu_ref[0],
                     preferred_element_type=jnp.float32) + gub_ref[0]
        gate = jnp.minimum(gu[:, :INTER], LIMIT)
        up = jnp.clip(gu[:, INTER:], -LIMIT, LIMIT)
        act = (up + 1.0) * (gate * jax.nn.sigmoid(gate * ALPHA))
        yd_ref[...] = jnp.dot(act, wd_ref[0],
                              preferred_element_type=jnp.float32) + db_ref[0]


def _grouped(meta, valid, xd, wgu, gub3, wd, db3):
    grid_spec = pltpu.PrefetchScalarGridSpec(
        num_scalar_prefetch=2,
        grid=(NT,),
        in_specs=[
            pl.BlockSpec((TM, H), lambda t, m, v: (t, 0)),
            pl.BlockSpec((1, H, 2 * INTER), lambda t, m, v: (m[t], 0, 0)),
            pl.BlockSpec((1, 1, 2 * INTER), lambda t, m, v: (m[t], 0, 0)),
            pl.BlockSpec((1, INTER, H), lambda t, m, v: (m[t], 0, 0)),
            pl.BlockSpec((1, 1, H), lambda t, m, v: (m[t], 0, 0)),
        ],
        out_specs=pl.BlockSpec((TM, H), lambda t, m, v: (t, 0)),
    )
    return pl.pallas_call(
        _grouped_body,
        grid_spec=grid_spec,
        out_shape=jax.ShapeDtypeStruct((ND, H), jnp.float32),
        compiler_params=pltpu.CompilerParams(
            dimension_semantics=("arbitrary",)),
    )(meta, valid, xd, wgu, gub3, wd, db3)


_CC = 16
_NC = TPW // _CC


def _sc_combine_body(yd_hbm, p1_hbm, p2_hbm, w1_hbm, w2_hbm, out_hbm,
                     pc1_v, pc2_v, wc1_v, wc2_v, rows1_v, rows2_v, obuf_v,
                     sem1, sem2):
    wid = lax.axis_index("s") * 2 + lax.axis_index("c")
    pltpu.sync_copy(p1_hbm.at[pl.ds(wid * _NC, _NC)], pc1_v)
    pltpu.sync_copy(p2_hbm.at[pl.ds(wid * _NC, _NC)], pc2_v)
    pltpu.sync_copy(w1_hbm.at[pl.ds(wid * _NC, _NC)], wc1_v)
    pltpu.sync_copy(w2_hbm.at[pl.ds(wid * _NC, _NC)], wc2_v)
    cps = {}
    for c in range(2):
        cps[c] = (pltpu.async_copy(yd_hbm.at[pc1_v.at[c]],
                                   rows1_v.at[c % 2], sem1),
                  pltpu.async_copy(yd_hbm.at[pc2_v.at[c]],
                                   rows2_v.at[c % 2], sem2))
    for c in range(_NC):
        b = c % 2
        cps[c][0].wait()
        cps[c][1].wait()
        wr1 = wc1_v[c, :]
        wr2 = wc2_v[c, :]
        for j in range(_CC):
            w1v = jnp.full((16,), wr1[j], jnp.float32)
            w2v = jnp.full((16,), wr2[j], jnp.float32)

            def body(i, _, b=b, j=j, w1v=w1v, w2v=w2v):
                for u in range(8):
                    sl = pl.ds(i * 128 + u * 16, 16)
                    obuf_v[j, sl] = (w1v * rows1_v[b, j, sl] +
                                     w2v * rows2_v[b, j, sl])
                return 0

            lax.fori_loop(0, H // 128, body, 0)
        pltpu.sync_copy(obuf_v,
                        out_hbm.at[pl.ds(wid * TPW + c * _CC, _CC)])
        if c + 2 < _NC:
            cps[c + 2] = (pltpu.async_copy(yd_hbm.at[pc1_v.at[c + 2]],
                                           rows1_v.at[b], sem1),
                          pltpu.async_copy(yd_hbm.at[pc2_v.at[c + 2]],
                                           rows2_v.at[b], sem2))


def _sc_combine(yd, p1, p2, w1, w2):
    return pl.kernel(
        _sc_combine_body,
        out_type=jax.ShapeDtypeStruct((T, H), jnp.float32),
        mesh=plsc.VectorSubcoreMesh(core_axis_name="c", subcore_axis_name="s",
                                    num_cores=2, num_subcores=16),
        scratch_types=[
            pltpu.VMEM((_NC, _CC), jnp.int32),
            pltpu.VMEM((_NC, _CC), jnp.int32),
            pltpu.VMEM((_NC, _CC), jnp.float32),
            pltpu.VMEM((_NC, _CC), jnp.float32),
            pltpu.VMEM((2, _CC, H), jnp.float32),
            pltpu.VMEM((2, _CC, H), jnp.float32),
            pltpu.VMEM((_CC, H), jnp.float32),
            pltpu.SemaphoreType.DMA,
            pltpu.SemaphoreType.DMA,
        ],
    )(yd, p1, p2, w1, w2)


def kernel(hidden_states, router_weight, router_bias, gate_up_proj,
           gate_up_bias, down_proj, down_bias):
    flat = hidden_states.reshape(T, H)
    rb2 = router_bias.reshape(E, 1)
    w1, w2, p1, p2, meta, valid = _router(flat, router_weight, rb2)
    xd = _sc_dispatch(p1.reshape(NW, TPW), p2.reshape(NW, TPW), flat)
    yd = _grouped(meta.reshape(NMETA), valid.reshape(NMETA), xd,
                  gate_up_proj, gate_up_bias.reshape(E, 1, 2 * INTER),
                  down_proj, down_bias.reshape(E, 1, H))
    out = _sc_combine(yd,
                      p1.reshape(NW * _NC, _CC), p2.reshape(NW * _NC, _CC),
                      w1.reshape(NW * _NC, _CC), w2.reshape(NW * _NC, _CC))
    return out.reshape(B, S, H)
